# baseline (speedup 1.0000x reference)
import functools
import math

import jax
import jax.numpy as jnp
from jax import lax
from jax.experimental import pallas as pl
from jax.experimental.pallas import tpu as pltpu
from jax.experimental.pallas import tpu_sc as plsc

_VOCAB = 1000000
_T = 200
_D = 64
_B = 4096

_NUM_CORES = 2
_NUM_SUBCORES = 16
_D_PER_CORE = _D // _NUM_CORES
_BW = _B // _NUM_SUBCORES
_TB = 25
_NBLK = _T // _TB

_SCALE = math.sqrt(_D)


def _make_pe_splat():
    pos = jnp.arange(_T, dtype=jnp.float32)[:, None]
    i = jnp.arange(0, _D, 2, dtype=jnp.float32)[None, :]
    angle = pos / jnp.power(10000.0, 2.0 * i / _D)
    pe = jnp.zeros((_T, _D), dtype=jnp.float32)
    pe = pe.at[:, 0::2].set(jnp.sin(angle))
    pe = pe.at[:, 1::2].set(jnp.cos(angle))
    return jnp.repeat(pe.T[:, :, None], 16, axis=2).reshape(_D, _T * 16)


def _sc_body(xT, tabT, peS, outT, xv, dst0, dst1, pe_v, ssem, psem, gsem,
             osem, xsem, spm):
    c = lax.axis_index("c")
    s = lax.axis_index("s")
    b0 = s * _BW
    dbase = c * _D_PER_CORE

    def tab_stage_desc(d):
        return pltpu.make_async_copy(tabT.at[d], spm, ssem)

    def pe_stage_desc(d):
        return pltpu.make_async_copy(peS.at[d], pe_v, psem)

    dsts = (dst0, dst1)

    def gather_desc(blk, p):
        return pltpu.make_async_copy(
            spm.at[xv.at[pl.ds(blk * _TB * _BW, _TB * _BW)]],
            dsts[p], gsem)

    def out_desc(t, d, p):
        return pltpu.make_async_copy(
            dsts[p].at[pl.ds((t % _TB) * _BW, _BW)],
            outT.at[t, d, pl.ds(b0, _BW)], osem)

    def xdesc(t):
        return pltpu.make_async_copy(
            xT.at[t, pl.ds(b0, _BW)], xv.at[pl.ds(t * _BW, _BW)], xsem)

    def xfire(t, _):
        xdesc(t).start()
        return 0

    def xdrain(t, _):
        xdesc(t).wait()
        return 0

    lax.fori_loop(0, _T, xfire, 0)
    lax.fori_loop(0, _T, xdrain, 0)
    pe_stage_desc(dbase).start()

    @pl.when(s == 0)
    def _():
        tab_stage_desc(dbase).start()
        tab_stage_desc(dbase).wait()

    pe_stage_desc(dbase).wait()
    plsc.subcore_barrier()

    def dbody(i, _):
        d = dbase + i

        def blockpair(bp, _):
            for p in range(2):
                blk = bp * 2 + p
                t0 = blk * _TB

                @pl.when(i * _NBLK + blk >= 2)
                def _(blk=blk, t0=t0, p=p):
                    tprev = (t0 - 2 * _TB) % _T
                    dprev = jnp.where(blk >= 2, d, d - 1)

                    def odrain(j, _):
                        out_desc(tprev + j, dprev, p).wait()
                        return 0

                    lax.fori_loop(0, _TB, odrain, 0)

                gather_desc(blk, p).start()

            for p in range(2):
                blk = bp * 2 + p
                t0 = blk * _TB
                gather_desc(blk, p).wait()

                def tbody(j, _, t0=t0, p=p):
                    t = t0 + j
                    off = j * _BW
                    pev = pe_v[pl.ds(t * 16, 16)]
                    for k in range(_BW // 16):
                        sl = pl.ds(off + k * 16, 16)
                        dsts[p][sl] = dsts[p][sl] * _SCALE + pev
                    out_desc(t, d, p).start()
                    return 0

                lax.fori_loop(0, _TB, tbody, 0)
            return 0

        lax.fori_loop(0, _NBLK // 2, blockpair, 0)

        plsc.subcore_barrier()

        @pl.when(i + 1 < _D_PER_CORE)
        def _():
            @pl.when(s == 0)
            def _():
                tab_stage_desc(d + 1).start()
                tab_stage_desc(d + 1).wait()

            pe_stage_desc(d + 1).start()
            pe_stage_desc(d + 1).wait()

        plsc.subcore_barrier()
        return 0

    lax.fori_loop(0, _D_PER_CORE, dbody, 0)

    dlast = dbase + _D_PER_CORE - 1
    for p in range(2):
        t0 = _T - 2 * _TB + p * _TB

        def odrain_last(j, _, t0=t0, p=p):
            out_desc(t0 + j, dlast, p).wait()
            return 0

        lax.fori_loop(0, _TB, odrain_last, 0)


@jax.jit
def _run(xT, tabT, peS):
    mesh = plsc.VectorSubcoreMesh(core_axis_name="c", subcore_axis_name="s")
    k = functools.partial(
        pl.kernel,
        mesh=mesh,
        out_type=jax.ShapeDtypeStruct((_T, _D, _B), jnp.float32),
        scratch_types=[
            pltpu.VMEM((_T * _BW,), jnp.int32),
            pltpu.VMEM((_TB * _BW,), jnp.float32),
            pltpu.VMEM((_TB * _BW,), jnp.float32),
            pltpu.VMEM((_T * 16,), jnp.float32),
            pltpu.SemaphoreType.DMA,
            pltpu.SemaphoreType.DMA,
            pltpu.SemaphoreType.DMA,
            pltpu.SemaphoreType.DMA,
            pltpu.SemaphoreType.DMA,
            pltpu.VMEM_SHARED((_VOCAB,), jnp.float32),
        ],
    )(_sc_body)
    return k(xT, tabT, peS)


def kernel(x, table):
    peS = _make_pe_splat()
    outT = _run(x.T, table.T, peS)
    return outT.transpose(2, 0, 1)


# speedup vs baseline: 1.0458x; 1.0458x over previous
import functools
import math

import jax
import jax.numpy as jnp
from jax import lax
from jax.experimental import pallas as pl
from jax.experimental.pallas import tpu as pltpu
from jax.experimental.pallas import tpu_sc as plsc

_VOCAB = 1000000
_T = 200
_D = 64
_B = 4096

_NUM_CORES = 2
_NUM_SUBCORES = 16
_D_PER_CORE = _D // _NUM_CORES
_BW = _B // _NUM_SUBCORES
_TB = 25
_NBLK = _T // _TB

_SCALE = math.sqrt(_D)


def _make_pe_splat():
    pos = jnp.arange(_T, dtype=jnp.float32)[:, None]
    i = jnp.arange(0, _D, 2, dtype=jnp.float32)[None, :]
    angle = pos / jnp.power(10000.0, 2.0 * i / _D)
    pe = jnp.zeros((_T, _D), dtype=jnp.float32)
    pe = pe.at[:, 0::2].set(jnp.sin(angle))
    pe = pe.at[:, 1::2].set(jnp.cos(angle))
    return jnp.repeat(pe.T[:, :, None], 16, axis=2).reshape(_D, _T * 16)


def _sc_body(xT, tabT, peS, outT, xv, dst0, dst1, pe_v, ssem, psem, gsem,
             osem, xsem, spm):
    c = lax.axis_index("c")
    s = lax.axis_index("s")
    b0 = s * _BW
    dbase = c * _D_PER_CORE

    def tab_stage_desc(d):
        return pltpu.make_async_copy(tabT.at[d], spm, ssem)

    def pe_stage_desc(d):
        return pltpu.make_async_copy(peS.at[d], pe_v, psem)

    dsts = (dst0, dst1)

    def gather_desc(blk, p):
        return pltpu.make_async_copy(
            spm.at[xv.at[pl.ds(blk * _TB * _BW, _TB * _BW)]],
            dsts[p], gsem)

    def out_desc(t, d, p):
        return pltpu.make_async_copy(
            dsts[p].at[pl.ds((t % _TB) * _BW, _BW)],
            outT.at[t, d, pl.ds(b0, _BW)], osem)

    def xdesc(t):
        return pltpu.make_async_copy(
            xT.at[t, pl.ds(b0, _BW)], xv.at[pl.ds(t * _BW, _BW)], xsem)

    def xfire(t, _):
        xdesc(t).start()
        return 0

    def xdrain(t, _):
        xdesc(t).wait()
        return 0

    lax.fori_loop(0, _T, xfire, 0)
    lax.fori_loop(0, _T, xdrain, 0)
    pe_stage_desc(dbase).start()

    @pl.when(s == 0)
    def _():
        tab_stage_desc(dbase).start()
        tab_stage_desc(dbase).wait()

    pe_stage_desc(dbase).wait()
    plsc.subcore_barrier()

    def dbody(i, _):
        d = dbase + i

        def blockpair(bp, _):
            for p in range(2):
                blk = bp * 2 + p
                t0 = blk * _TB

                @pl.when(i * _NBLK + blk >= 2)
                def _(blk=blk, t0=t0, p=p):
                    tprev = (t0 - 2 * _TB) % _T
                    dprev = jnp.where(blk >= 2, d, d - 1)

                    def odrain(j, _):
                        out_desc(tprev + j, dprev, p).wait()
                        return 0

                    lax.fori_loop(0, _TB, odrain, 0)

                gather_desc(blk, p).start()

            for p in range(2):
                blk = bp * 2 + p
                t0 = blk * _TB
                gather_desc(blk, p).wait()

                def tbody(j, _, t0=t0, p=p):
                    t = t0 + j
                    off = j * _BW
                    pev = pe_v[pl.ds(t * 16, 16)]
                    for k in range(_BW // 16):
                        sl = pl.ds(off + k * 16, 16)
                        dsts[p][sl] = dsts[p][sl] * _SCALE + pev
                    out_desc(t, d, p).start()
                    return 0

                lax.fori_loop(0, _TB, tbody, 0)
            return 0

        lax.fori_loop(0, _NBLK // 2 - 1, blockpair, 0)

        bp = _NBLK // 2 - 1
        for p in range(2):
            blk = bp * 2 + p
            t0 = blk * _TB
            tprev = t0 - 2 * _TB

            def odrain(j, _, tprev=tprev, p=p):
                out_desc(tprev + j, d, p).wait()
                return 0

            lax.fori_loop(0, _TB, odrain, 0)
            gather_desc(blk, p).start()

        for p in range(2):
            gather_desc(bp * 2 + p, p).wait()

        plsc.subcore_barrier()

        @pl.when((i + 1 < _D_PER_CORE) & (s == 0))
        def _():
            tab_stage_desc(d + 1).start()

        for p in range(2):
            blk = bp * 2 + p
            t0 = blk * _TB

            def tbody2(j, _, t0=t0, p=p):
                t = t0 + j
                off = j * _BW
                pev = pe_v[pl.ds(t * 16, 16)]
                for k in range(_BW // 16):
                    sl = pl.ds(off + k * 16, 16)
                    dsts[p][sl] = dsts[p][sl] * _SCALE + pev
                out_desc(t, d, p).start()
                return 0

            lax.fori_loop(0, _TB, tbody2, 0)

        @pl.when(i + 1 < _D_PER_CORE)
        def _():
            pe_stage_desc(d + 1).start()
            pe_stage_desc(d + 1).wait()

            @pl.when(s == 0)
            def _():
                tab_stage_desc(d + 1).wait()

        plsc.subcore_barrier()
        return 0

    lax.fori_loop(0, _D_PER_CORE, dbody, 0)

    dlast = dbase + _D_PER_CORE - 1
    for p in range(2):
        t0 = _T - 2 * _TB + p * _TB

        def odrain_last(j, _, t0=t0, p=p):
            out_desc(t0 + j, dlast, p).wait()
            return 0

        lax.fori_loop(0, _TB, odrain_last, 0)


@jax.jit
def _run(xT, tabT, peS):
    mesh = plsc.VectorSubcoreMesh(core_axis_name="c", subcore_axis_name="s")
    k = functools.partial(
        pl.kernel,
        mesh=mesh,
        out_type=jax.ShapeDtypeStruct((_T, _D, _B), jnp.float32),
        scratch_types=[
            pltpu.VMEM((_T * _BW,), jnp.int32),
            pltpu.VMEM((_TB * _BW,), jnp.float32),
            pltpu.VMEM((_TB * _BW,), jnp.float32),
            pltpu.VMEM((_T * 16,), jnp.float32),
            pltpu.SemaphoreType.DMA,
            pltpu.SemaphoreType.DMA,
            pltpu.SemaphoreType.DMA,
            pltpu.SemaphoreType.DMA,
            pltpu.SemaphoreType.DMA,
            pltpu.VMEM_SHARED((_VOCAB,), jnp.float32),
        ],
    )(_sc_body)
    return k(xT, tabT, peS)


def kernel(x, table):
    peS = _make_pe_splat()
    outT = _run(x.T, table.T, peS)
    return outT.transpose(2, 0, 1)


# speedup vs baseline: 1.0472x; 1.0014x over previous
import functools
import math

import jax
import jax.numpy as jnp
from jax import lax
from jax.experimental import pallas as pl
from jax.experimental.pallas import tpu as pltpu
from jax.experimental.pallas import tpu_sc as plsc

_VOCAB = 1000000
_T = 200
_D = 64
_B = 4096

_NUM_CORES = 2
_NUM_SUBCORES = 16
_D_PER_CORE = _D // _NUM_CORES
_BW = _B // _NUM_SUBCORES
_TB = 25
_NBLK = _T // _TB

_SCALE = math.sqrt(_D)


def _make_pe_splat():
    pos = jnp.arange(_T, dtype=jnp.float32)[:, None]
    i = jnp.arange(0, _D, 2, dtype=jnp.float32)[None, :]
    angle = pos / jnp.power(10000.0, 2.0 * i / _D)
    pe = jnp.zeros((_T, _D), dtype=jnp.float32)
    pe = pe.at[:, 0::2].set(jnp.sin(angle))
    pe = pe.at[:, 1::2].set(jnp.cos(angle))
    return jnp.repeat(pe.T[:, :, None], 16, axis=2).reshape(_D, _T * 16)


def _sc_body(xT, tabT, peS, outT, xv, dst0, dst1, pe_v, ssem, psem, gsem,
             osem, xsem, spm):
    c = lax.axis_index("c")
    s = lax.axis_index("s")
    b0 = s * _BW
    dbase = c * _D_PER_CORE

    def tab_stage_desc(d):
        return pltpu.make_async_copy(tabT.at[d], spm, ssem)

    def pe_stage_desc(d):
        return pltpu.make_async_copy(peS.at[d], pe_v, psem)

    dsts = (dst0, dst1)

    _GH = _TB * _BW // 2

    def gather_descs(blk, p):
        base = blk * _TB * _BW
        return [
            pltpu.make_async_copy(
                spm.at[xv.at[pl.ds(base + h * _GH, _GH)]],
                dsts[p].at[pl.ds(h * _GH, _GH)], gsem)
            for h in range(2)
        ]

    def out_desc(t, d, p):
        return pltpu.make_async_copy(
            dsts[p].at[pl.ds((t % _TB) * _BW, _BW)],
            outT.at[t, d, pl.ds(b0, _BW)], osem)

    def xdesc(t):
        return pltpu.make_async_copy(
            xT.at[t, pl.ds(b0, _BW)], xv.at[pl.ds(t * _BW, _BW)], xsem)

    def xfire(t, _):
        xdesc(t).start()
        return 0

    def xdrain(t, _):
        xdesc(t).wait()
        return 0

    lax.fori_loop(0, _T, xfire, 0)
    lax.fori_loop(0, _T, xdrain, 0)
    pe_stage_desc(dbase).start()

    @pl.when(s == 0)
    def _():
        tab_stage_desc(dbase).start()
        tab_stage_desc(dbase).wait()

    pe_stage_desc(dbase).wait()
    plsc.subcore_barrier()

    def dbody(i, _):
        d = dbase + i

        def blockpair(bp, _):
            for p in range(2):
                blk = bp * 2 + p
                t0 = blk * _TB

                @pl.when(i * _NBLK + blk >= 2)
                def _(blk=blk, t0=t0, p=p):
                    tprev = (t0 - 2 * _TB) % _T
                    dprev = jnp.where(blk >= 2, d, d - 1)

                    def odrain(j, _):
                        out_desc(tprev + j, dprev, p).wait()
                        return 0

                    lax.fori_loop(0, _TB, odrain, 0)

                for g in gather_descs(blk, p):
                    g.start()

            for p in range(2):
                blk = bp * 2 + p
                t0 = blk * _TB
                for g in gather_descs(blk, p):
                    g.wait()

                def tbody(j, _, t0=t0, p=p):
                    t = t0 + j
                    off = j * _BW
                    pev = pe_v[pl.ds(t * 16, 16)]
                    for k in range(_BW // 16):
                        sl = pl.ds(off + k * 16, 16)
                        dsts[p][sl] = dsts[p][sl] * _SCALE + pev
                    out_desc(t, d, p).start()
                    return 0

                lax.fori_loop(0, _TB, tbody, 0)
            return 0

        lax.fori_loop(0, _NBLK // 2 - 1, blockpair, 0)

        bp = _NBLK // 2 - 1
        for p in range(2):
            blk = bp * 2 + p
            t0 = blk * _TB
            tprev = t0 - 2 * _TB

            def odrain(j, _, tprev=tprev, p=p):
                out_desc(tprev + j, d, p).wait()
                return 0

            lax.fori_loop(0, _TB, odrain, 0)
            for g in gather_descs(blk, p):
                g.start()

        for p in range(2):
            for g in gather_descs(bp * 2 + p, p):
                g.wait()

        plsc.subcore_barrier()

        @pl.when((i + 1 < _D_PER_CORE) & (s == 0))
        def _():
            tab_stage_desc(d + 1).start()

        for p in range(2):
            blk = bp * 2 + p
            t0 = blk * _TB

            def tbody2(j, _, t0=t0, p=p):
                t = t0 + j
                off = j * _BW
                pev = pe_v[pl.ds(t * 16, 16)]
                for k in range(_BW // 16):
                    sl = pl.ds(off + k * 16, 16)
                    dsts[p][sl] = dsts[p][sl] * _SCALE + pev
                out_desc(t, d, p).start()
                return 0

            lax.fori_loop(0, _TB, tbody2, 0)

        @pl.when(i + 1 < _D_PER_CORE)
        def _():
            pe_stage_desc(d + 1).start()
            pe_stage_desc(d + 1).wait()

            @pl.when(s == 0)
            def _():
                tab_stage_desc(d + 1).wait()

        plsc.subcore_barrier()
        return 0

    lax.fori_loop(0, _D_PER_CORE, dbody, 0)

    dlast = dbase + _D_PER_CORE - 1
    for p in range(2):
        t0 = _T - 2 * _TB + p * _TB

        def odrain_last(j, _, t0=t0, p=p):
            out_desc(t0 + j, dlast, p).wait()
            return 0

        lax.fori_loop(0, _TB, odrain_last, 0)


@jax.jit
def _run(xT, tabT, peS):
    mesh = plsc.VectorSubcoreMesh(core_axis_name="c", subcore_axis_name="s")
    k = functools.partial(
        pl.kernel,
        mesh=mesh,
        out_type=jax.ShapeDtypeStruct((_T, _D, _B), jnp.float32),
        scratch_types=[
            pltpu.VMEM((_T * _BW,), jnp.int32),
            pltpu.VMEM((_TB * _BW,), jnp.float32),
            pltpu.VMEM((_TB * _BW,), jnp.float32),
            pltpu.VMEM((_T * 16,), jnp.float32),
            pltpu.SemaphoreType.DMA,
            pltpu.SemaphoreType.DMA,
            pltpu.SemaphoreType.DMA,
            pltpu.SemaphoreType.DMA,
            pltpu.SemaphoreType.DMA,
            pltpu.VMEM_SHARED((_VOCAB,), jnp.float32),
        ],
    )(_sc_body)
    return k(xT, tabT, peS)


def kernel(x, table):
    peS = _make_pe_splat()
    outT = _run(x.T, table.T, peS)
    return outT.transpose(2, 0, 1)
